# baseline (speedup 1.0000x reference)
import functools

import jax
import jax.numpy as jnp
from jax import lax
from jax.experimental import pallas as pl
from jax.experimental.pallas import tpu as pltpu
from jax.experimental.pallas import tpu_sc as plsc

NC = 2
NS = 16
NW = NC * NS
K = 100
NBUF = 4


def _mesh():
    return plsc.VectorSubcoreMesh(
        core_axis_name="c", subcore_axis_name="s",
        num_cores=NC, num_subcores=NS)


def _pad_n(n):
    return ((n + 8 * NS - 1) // (8 * NS)) * (8 * NS)


def _sc_degree(col3, n):
    nch = col3.shape[1]
    np_ = _pad_n(n)
    stripe = np_ // NS

    @functools.partial(
        pl.kernel,
        out_type=jax.ShapeDtypeStruct((NC, np_, 16), jnp.float32),
        mesh=_mesh(),
        compiler_params=pltpu.CompilerParams(use_tc_tiling_on_sc=False),
        scratch_types=[
            pltpu.VMEM((nch, K), jnp.int32),
            pltpu.VMEM((K, 16), jnp.float32),
            pltpu.VMEM_SHARED((np_, 16), jnp.float32),
            [pltpu.SemaphoreType.DMA] * NBUF,
        ],
    )
    def deg_kernel(col_hbm, ones_hbm, zeros_hbm, out_hbm, cidx, ones_v, acc,
                   sems):
        cid = lax.axis_index("c")
        sid = lax.axis_index("s")
        wid = sid * NC + cid
        pltpu.sync_copy(zeros_hbm.at[pl.ds(sid * stripe, stripe)],
                        acc.at[pl.ds(sid * stripe, stripe)])
        pltpu.sync_copy(ones_hbm, ones_v)
        pltpu.sync_copy(col_hbm.at[wid], cidx)
        plsc.subcore_barrier()

        def scat(j, b):
            return pltpu.make_async_copy(ones_v, acc.at[cidx.at[j]], sems[b])

        @pl.loop(0, nch, step=NBUF)
        def _(q):
            for b in range(NBUF):
                @pl.when(q > 0)
                def _():
                    scat(q - NBUF + b, b).wait()
                scat(q + b, b).start(add=True)

        for b in range(NBUF):
            scat(nch - NBUF + b, b).wait()
        plsc.subcore_barrier()
        pltpu.sync_copy(acc.at[pl.ds(sid * stripe, stripe)],
                        out_hbm.at[cid].at[pl.ds(sid * stripe, stripe)])

    return deg_kernel(col3, jnp.ones((K, 16), jnp.float32),
                      jnp.zeros((np_, 16), jnp.float32))


def _sc_scatter(row2, col2, table2, zeros_h):
    _, np_, dh = table2.shape
    nch = row2.shape[1]
    stripe = np_ // NS
    assert table2.shape[0] == 4

    @functools.partial(
        pl.kernel,
        out_type=jax.ShapeDtypeStruct((NC, np_, dh), jnp.float32),
        mesh=_mesh(),
        compiler_params=pltpu.CompilerParams(use_tc_tiling_on_sc=False),
        scratch_types=[
            pltpu.VMEM((nch, K), jnp.int32),
            pltpu.VMEM((nch, K), jnp.int32),
            pltpu.VMEM((NBUF, K, dh), jnp.float32),
            pltpu.VMEM_SHARED((np_, dh), jnp.float32),
            [pltpu.SemaphoreType.DMA] * NBUF,
            [pltpu.SemaphoreType.DMA] * NBUF,
        ],
    )
    def edge_kernel(row_hbm, col_hbm, table_hbm, zeros_hbm, out_hbm,
                    ridx, cidx, rows, acc, sem_g, sem_s):
        cid = lax.axis_index("c")
        sid = lax.axis_index("s")
        sl = pl.ds(sid * stripe, stripe)
        pltpu.sync_copy(row_hbm.at[sid], ridx)
        pltpu.sync_copy(col_hbm.at[sid], cidx)
        pltpu.sync_copy(zeros_hbm.at[sl], acc.at[sl])
        plsc.subcore_barrier()

        def gat(j, b):
            return pltpu.make_async_copy(table_hbm.at[cid].at[ridx.at[j]],
                                         rows.at[b], sem_g[b])

        def scat(j, b):
            return pltpu.make_async_copy(rows.at[b], acc.at[cidx.at[j]],
                                         sem_s[b])

        @pl.loop(0, nch, step=NBUF)
        def _(q):
            for b in range(NBUF):
                c = q + b

                @pl.when(q > 0)
                def _():
                    scat(c - NBUF, b).wait()
                gat(c, b).start()

                b2 = (b + 2) % NBUF
                if b >= 2:
                    gat(c - 2, b2).wait()
                    scat(c - 2, b2).start(add=True)
                else:
                    @pl.when(q > 0)
                    def _():
                        gat(c - 2, b2).wait()
                        scat(c - 2, b2).start(add=True)

        for c in (nch - 2, nch - 1):
            b = c % NBUF
            gat(c, b).wait()
            scat(c, b).start(add=True)
        for c in range(nch - NBUF, nch):
            scat(c, c % NBUF).wait()
        plsc.subcore_barrier()
        pltpu.sync_copy(acc.at[sl], out_hbm.at[cid].at[sl])

    return edge_kernel(row2, col2, table2, zeros_h)


def _split_halves(o_ref, val, n, np_, dh):
    o_ref[0, :n, :] = val[:, :dh]
    o_ref[1, :n, :] = val[:, dh:]
    pad = jnp.zeros((np_ - n, dh), jnp.float32)
    o_ref[0, n:, :] = pad
    o_ref[1, n:, :] = pad


def _tc_prep(degp, x, w0):
    n = x.shape[0]
    d = w0.shape[1]
    np_ = degp.shape[1]
    dh = d // 2

    def body(dp_ref, x_ref, w_ref, dis_ref, t2_ref):
        deg1 = 1.0 + dp_ref[0, :n, :1] + dp_ref[1, :n, :1]
        dis1 = lax.rsqrt(deg1)
        dis_ref[...] = dis1
        hw = jnp.dot(x_ref[...], w_ref[...], preferred_element_type=jnp.float32)
        _split_halves(t2_ref, hw * dis1, n, np_, dh)

    return pl.pallas_call(
        body,
        out_shape=[jax.ShapeDtypeStruct((n, 1), jnp.float32),
                   jax.ShapeDtypeStruct((2, np_, dh), jnp.float32)],
    )(degp, x, w0)


def _tc_layer(p, t2_prev, dis, b, w_next):
    _, np_, dh = p.shape
    n = dis.shape[0]

    def body(p_ref, t2p_ref, dis_ref, b_ref, w_ref, o_ref):
        dis1 = dis_ref[...]
        s = jnp.concatenate(
            [p_ref[0, :n, :] + t2p_ref[0, :n, :],
             p_ref[1, :n, :] + t2p_ref[1, :n, :]], axis=1)
        h = jnp.maximum(s * dis1 + b_ref[...], 0.0)
        hw = jnp.dot(h, w_ref[...], preferred_element_type=jnp.float32)
        _split_halves(o_ref, hw * dis1, n, np_, dh)

    return pl.pallas_call(
        body,
        out_shape=jax.ShapeDtypeStruct((2, np_, dh), jnp.float32),
    )(p, t2_prev, dis, b, w_next)


def _tc_head(p, t2_prev, dis, b2, wo1, bo1, gamma, beta, wo2, bo2):
    _, np_, dh = p.shape
    n = dis.shape[0]
    inv_bn = (1.0 + 1e-5) ** -0.5

    def body(p_ref, t2p_ref, dis_ref, b2_ref, w1_ref, b1_ref, g_ref, be_ref,
             w2_ref, bo2_ref, o_ref):
        dis1 = dis_ref[...]
        s = jnp.concatenate(
            [p_ref[0, :n, :] + t2p_ref[0, :n, :],
             p_ref[1, :n, :] + t2p_ref[1, :n, :]], axis=1)
        h = jnp.maximum(s * dis1 + b2_ref[...], 0.0)
        t = jnp.dot(h, w1_ref[...], preferred_element_type=jnp.float32)
        t = t + b1_ref[...]
        t = t * (g_ref[...] * inv_bn) + be_ref[...]
        t = jnp.maximum(t, 0.0)
        o = jnp.dot(t, w2_ref[...], preferred_element_type=jnp.float32)
        o_ref[...] = o + bo2_ref[...]

    return pl.pallas_call(
        body,
        out_shape=jax.ShapeDtypeStruct((n, wo2.shape[1]), jnp.float32),
    )(p, t2_prev, dis, b2, wo1, bo1, gamma, beta, wo2, bo2)


def kernel(x, edge_index, W0, b0, W1, b1, W2, b2, Wo1, bo1, gamma, beta,
           Wo2, bo2):
    n, d = x.shape
    e = edge_index.shape[1]
    np_ = _pad_n(n)
    dh = d // 2
    nch_deg = e // (NW * K)
    nch = e // (NS * K)
    row2 = edge_index[0].reshape(NS, nch, K)
    col2 = edge_index[1].reshape(NS, nch, K)
    col3 = edge_index[1].reshape(NW, nch_deg, K)
    b0r = b0.reshape(1, -1)
    b1r = b1.reshape(1, -1)
    b2r = b2.reshape(1, -1)
    bo1r = bo1.reshape(1, -1)
    gr = gamma.reshape(1, -1)
    ber = beta.reshape(1, -1)
    bo2r = bo2.reshape(1, -1)
    zeros_h = jnp.zeros((np_, dh), jnp.float32)

    degp = _sc_degree(col3, n)
    dis, t2_0 = _tc_prep(degp, x, W0)

    p0 = _sc_scatter(row2, col2, jnp.tile(t2_0, (2, 1, 1)), zeros_h)
    t2_1 = _tc_layer(p0, t2_0, dis, b0r, W1)
    p1 = _sc_scatter(row2, col2, jnp.tile(t2_1, (2, 1, 1)), zeros_h)
    t2_2 = _tc_layer(p1, t2_1, dis, b1r, W2)
    p2 = _sc_scatter(row2, col2, jnp.tile(t2_2, (2, 1, 1)), zeros_h)
    return _tc_head(p2, t2_2, dis, b2r, Wo1, bo1r, gr, ber, Wo2, bo2r)


# speedup vs baseline: 1.1288x; 1.1288x over previous
import functools

import jax
import jax.numpy as jnp
from jax import lax
from jax.experimental import pallas as pl
from jax.experimental.pallas import tpu as pltpu
from jax.experimental.pallas import tpu_sc as plsc

NC = 2
NS = 16
NW = NC * NS
K = 125
NBUF = 4


def _mesh():
    return plsc.VectorSubcoreMesh(
        core_axis_name="c", subcore_axis_name="s",
        num_cores=NC, num_subcores=NS)


def _pad_n(n):
    return ((n + 8 * NS - 1) // (8 * NS)) * (8 * NS)


def _sc_degree(col3, n):
    nch = col3.shape[1]
    np_ = _pad_n(n)
    stripe = np_ // NS

    @functools.partial(
        pl.kernel,
        out_type=jax.ShapeDtypeStruct((NC, np_, 16), jnp.float32),
        mesh=_mesh(),
        compiler_params=pltpu.CompilerParams(use_tc_tiling_on_sc=False),
        scratch_types=[
            pltpu.VMEM((nch, K), jnp.int32),
            pltpu.VMEM((K, 16), jnp.float32),
            pltpu.VMEM_SHARED((np_, 16), jnp.float32),
            [pltpu.SemaphoreType.DMA] * NBUF,
        ],
    )
    def deg_kernel(col_hbm, ones_hbm, zeros_hbm, out_hbm, cidx, ones_v, acc,
                   sems):
        cid = lax.axis_index("c")
        sid = lax.axis_index("s")
        wid = sid * NC + cid
        pltpu.sync_copy(zeros_hbm.at[pl.ds(sid * stripe, stripe)],
                        acc.at[pl.ds(sid * stripe, stripe)])
        pltpu.sync_copy(ones_hbm, ones_v)
        pltpu.sync_copy(col_hbm.at[wid], cidx)
        plsc.subcore_barrier()

        def scat(j, b):
            return pltpu.make_async_copy(ones_v, acc.at[cidx.at[j]], sems[b])

        @pl.loop(0, nch, step=NBUF)
        def _(q):
            for b in range(NBUF):
                @pl.when(q > 0)
                def _():
                    scat(q - NBUF + b, b).wait()
                scat(q + b, b).start(add=True)

        for b in range(NBUF):
            scat(nch - NBUF + b, b).wait()
        plsc.subcore_barrier()
        pltpu.sync_copy(acc.at[pl.ds(sid * stripe, stripe)],
                        out_hbm.at[cid].at[pl.ds(sid * stripe, stripe)])

    return deg_kernel(col3, jnp.ones((K, 16), jnp.float32),
                      jnp.zeros((np_, 16), jnp.float32))


def _sc_scatter(row2, col2, table2, zeros_h):
    _, np_, dh = table2.shape
    nch = row2.shape[1]
    stripe = np_ // NS

    @functools.partial(
        pl.kernel,
        out_type=jax.ShapeDtypeStruct((NC, np_, dh), jnp.float32),
        mesh=_mesh(),
        compiler_params=pltpu.CompilerParams(use_tc_tiling_on_sc=False),
        scratch_types=[
            pltpu.VMEM((nch, K), jnp.int32),
            pltpu.VMEM((nch, K), jnp.int32),
            pltpu.VMEM((NBUF, K, dh), jnp.float32),
            pltpu.VMEM_SHARED((np_, dh), jnp.float32),
            [pltpu.SemaphoreType.DMA] * NBUF,
            [pltpu.SemaphoreType.DMA] * NBUF,
        ],
    )
    def edge_kernel(row_hbm, col_hbm, table_hbm, zeros_hbm, out_hbm,
                    ridx, cidx, rows, acc, sem_g, sem_s):
        cid = lax.axis_index("c")
        sid = lax.axis_index("s")
        sl = pl.ds(sid * stripe, stripe)
        pltpu.sync_copy(row_hbm.at[sid], ridx)
        pltpu.sync_copy(col_hbm.at[sid], cidx)
        pltpu.sync_copy(zeros_hbm.at[sl], acc.at[sl])
        plsc.subcore_barrier()

        def gat(j, b):
            return pltpu.make_async_copy(table_hbm.at[cid].at[ridx.at[j]],
                                         rows.at[b], sem_g[b])

        def scat(j, b):
            return pltpu.make_async_copy(rows.at[b], acc.at[cidx.at[j]],
                                         sem_s[b])

        @pl.loop(0, nch, step=NBUF)
        def _(q):
            for b in range(NBUF):
                c = q + b

                @pl.when(q > 0)
                def _():
                    scat(c - NBUF, b).wait()
                gat(c, b).start()

                b2 = (b + 2) % NBUF
                if b >= 2:
                    gat(c - 2, b2).wait()
                    scat(c - 2, b2).start(add=True)
                else:
                    @pl.when(q > 0)
                    def _():
                        gat(c - 2, b2).wait()
                        scat(c - 2, b2).start(add=True)

        for c in (nch - 2, nch - 1):
            b = c % NBUF
            gat(c, b).wait()
            scat(c, b).start(add=True)
        for c in range(nch - NBUF, nch):
            scat(c, c % NBUF).wait()
        plsc.subcore_barrier()
        pltpu.sync_copy(acc.at[sl], out_hbm.at[cid].at[sl])

    return edge_kernel(row2, col2, table2, zeros_h)


def _split_halves(o_ref, val, n, np_, dh):
    o_ref[0, :n, :] = val[:, :dh]
    o_ref[1, :n, :] = val[:, dh:]
    pad = jnp.zeros((np_ - n, dh), jnp.float32)
    o_ref[0, n:, :] = pad
    o_ref[1, n:, :] = pad


def _tc_prep(degp, x, w0):
    n = x.shape[0]
    d = w0.shape[1]
    np_ = degp.shape[1]
    dh = d // 2

    def body(dp_ref, x_ref, w_ref, dis_ref, t2_ref):
        deg1 = 1.0 + dp_ref[0, :n, :1] + dp_ref[1, :n, :1]
        dis1 = lax.rsqrt(deg1)
        dis_ref[...] = dis1
        hw = jnp.dot(x_ref[...], w_ref[...], preferred_element_type=jnp.float32)
        _split_halves(t2_ref, hw * dis1, n, np_, dh)

    return pl.pallas_call(
        body,
        out_shape=[jax.ShapeDtypeStruct((n, 1), jnp.float32),
                   jax.ShapeDtypeStruct((2, np_, dh), jnp.float32)],
    )(degp, x, w0)


def _tc_layer(p, t2_prev, dis, b, w_next):
    _, np_, dh = p.shape
    n = dis.shape[0]

    def body(p_ref, t2p_ref, dis_ref, b_ref, w_ref, o_ref):
        dis1 = dis_ref[...]
        s = jnp.concatenate(
            [p_ref[0, :n, :] + t2p_ref[0, :n, :],
             p_ref[1, :n, :] + t2p_ref[1, :n, :]], axis=1)
        h = jnp.maximum(s * dis1 + b_ref[...], 0.0)
        hw = jnp.dot(h, w_ref[...], preferred_element_type=jnp.float32)
        _split_halves(o_ref, hw * dis1, n, np_, dh)

    return pl.pallas_call(
        body,
        out_shape=jax.ShapeDtypeStruct((2, np_, dh), jnp.float32),
    )(p, t2_prev, dis, b, w_next)


def _tc_head(p, t2_prev, dis, b2, wo1, bo1, gamma, beta, wo2, bo2):
    _, np_, dh = p.shape
    n = dis.shape[0]
    inv_bn = (1.0 + 1e-5) ** -0.5

    def body(p_ref, t2p_ref, dis_ref, b2_ref, w1_ref, b1_ref, g_ref, be_ref,
             w2_ref, bo2_ref, o_ref):
        dis1 = dis_ref[...]
        s = jnp.concatenate(
            [p_ref[0, :n, :] + t2p_ref[0, :n, :],
             p_ref[1, :n, :] + t2p_ref[1, :n, :]], axis=1)
        h = jnp.maximum(s * dis1 + b2_ref[...], 0.0)
        t = jnp.dot(h, w1_ref[...], preferred_element_type=jnp.float32)
        t = t + b1_ref[...]
        t = t * (g_ref[...] * inv_bn) + be_ref[...]
        t = jnp.maximum(t, 0.0)
        o = jnp.dot(t, w2_ref[...], preferred_element_type=jnp.float32)
        o_ref[...] = o + bo2_ref[...]

    return pl.pallas_call(
        body,
        out_shape=jax.ShapeDtypeStruct((n, wo2.shape[1]), jnp.float32),
    )(p, t2_prev, dis, b2, wo1, bo1, gamma, beta, wo2, bo2)


def kernel(x, edge_index, W0, b0, W1, b1, W2, b2, Wo1, bo1, gamma, beta,
           Wo2, bo2):
    n, d = x.shape
    e = edge_index.shape[1]
    np_ = _pad_n(n)
    dh = d // 2
    nch_deg = e // (NW * K)
    nch = e // (NS * K)
    row2 = edge_index[0].reshape(NS, nch, K)
    col2 = edge_index[1].reshape(NS, nch, K)
    col3 = edge_index[1].reshape(NW, nch_deg, K)
    b0r = b0.reshape(1, -1)
    b1r = b1.reshape(1, -1)
    b2r = b2.reshape(1, -1)
    bo1r = bo1.reshape(1, -1)
    gr = gamma.reshape(1, -1)
    ber = beta.reshape(1, -1)
    bo2r = bo2.reshape(1, -1)
    zeros_h = jnp.zeros((np_, dh), jnp.float32)

    degp = _sc_degree(col3, n)
    dis, t2_0 = _tc_prep(degp, x, W0)

    p0 = _sc_scatter(row2, col2, t2_0, zeros_h)
    t2_1 = _tc_layer(p0, t2_0, dis, b0r, W1)
    p1 = _sc_scatter(row2, col2, t2_1, zeros_h)
    t2_2 = _tc_layer(p1, t2_1, dis, b1r, W2)
    p2 = _sc_scatter(row2, col2, t2_2, zeros_h)
    return _tc_head(p2, t2_2, dis, b2r, Wo1, bo1r, gr, ber, Wo2, bo2r)
